# baseline (speedup 1.0000x reference)
import functools

import numpy as np
import jax
import jax.numpy as jnp
from jax import lax
from jax.experimental import pallas as pl
from jax.experimental.pallas import tpu as pltpu
from jax.experimental.pallas import tpu_sc as plsc

_N_FFT = 1024
_HOP = 256
_PAD = _N_FFT // 2
_EPAD = 256
_TOKPAD = 64

_n = np.arange(_N_FFT)
_w = (0.5 - 0.5 * np.cos(2.0 * np.pi * _n / _N_FFT)).astype(np.float32)
_walt = (_w * np.where(_n % 2 == 0, 1.0, -1.0)).astype(np.float32)
_wsq = (_w * _w).astype(np.float32)
_WMAT = np.zeros((16, _HOP), np.float32)
_WMAT[0:4] = _w.reshape(4, _HOP)
_WMAT[4:8] = _walt.reshape(4, _HOP)
_WMAT[8:12] = _wsq.reshape(4, _HOP)


def _energy_kernel(x_ref, xb_ref, w_ref, wr_ref, out_ref):
    x = x_ref[:]
    xb = xb_ref[:]
    w = w_ref[:]
    wr = wr_ref[:]
    B, n_interior, _ = x.shape
    n_chunks = n_interior + 4
    n_frames = n_chunks - 3

    xsq = x * x
    xbsq = xb * xb
    a = []
    for j in range(4):
        wj = w[j:j + 1, :].reshape(1, 1, _HOP)
        wjalt = w[j + 4:j + 5, :].reshape(1, 1, _HOP)
        wjsq = w[j + 8:j + 9, :].reshape(1, 1, _HOP)
        rj = wr[j:j + 1, :].reshape(1, 1, _HOP)
        rjalt = wr[j + 4:j + 5, :].reshape(1, 1, _HOP)
        rjsq = wr[j + 8:j + 9, :].reshape(1, 1, _HOP)
        ai = jnp.sum(x * wj, axis=-1)
        bi = jnp.sum(x * wjalt, axis=-1)
        ci = jnp.sum(xsq * wjsq, axis=-1)
        ab = jnp.sum(xb * rj, axis=-1)
        bb = jnp.sum(xb * rjalt, axis=-1)
        cb = jnp.sum(xbsq * rjsq, axis=-1)
        a.append((
            jnp.concatenate([ab[:, 1:2], ab[:, 0:1], ai,
                             ab[:, 3:4], ab[:, 2:3]], axis=1),
            jnp.concatenate([bb[:, 1:2], bb[:, 0:1], bi,
                             bb[:, 3:4], bb[:, 2:3]], axis=1),
            jnp.concatenate([cb[:, 1:2], cb[:, 0:1], ci,
                             cb[:, 3:4], cb[:, 2:3]], axis=1),
        ))

    s0 = a[0][0][:, 0:n_frames] + a[1][0][:, 1:n_frames + 1] \
        + a[2][0][:, 2:n_frames + 2] + a[3][0][:, 3:n_frames + 3]
    s1 = a[0][1][:, 0:n_frames] + a[1][1][:, 1:n_frames + 1] \
        + a[2][1][:, 2:n_frames + 2] + a[3][1][:, 3:n_frames + 3]
    s2 = a[0][2][:, 0:n_frames] + a[1][2][:, 1:n_frames + 1] \
        + a[2][2][:, 2:n_frames + 2] + a[3][2][:, 3:n_frames + 3]

    power = 0.5 * (_N_FFT * s2 + s0 * s0 + s1 * s1)
    e = jnp.sqrt(jnp.maximum(power, 1e-10))

    out_ref[:] = jnp.zeros((B, _EPAD), jnp.float32)
    out_ref[:, 0:n_frames] = e


def _make_sc_segment_mean(B, n_tok):
    mesh = plsc.VectorSubcoreMesh(core_axis_name="c", subcore_axis_name="s")

    @functools.partial(
        pl.kernel, mesh=mesh,
        out_type=jax.ShapeDtypeStruct((B, _TOKPAD), jnp.float32),
        compiler_params=pltpu.CompilerParams(
            needs_layout_passes=False,
            disable_bounds_checks=True,
            disable_semaphore_checks=True,
        ),
        scratch_types=[
            pltpu.VMEM((_EPAD,), jnp.float32),
            pltpu.VMEM((_EPAD,), jnp.float32),
            pltpu.VMEM((_EPAD,), jnp.float32),
            pltpu.VMEM((_TOKPAD,), jnp.int32),
            pltpu.VMEM((_TOKPAD,), jnp.int32),
            pltpu.VMEM((_TOKPAD,), jnp.float32),
            pltpu.SemaphoreType.DMA,
        ],
    )
    def sc_kernel(e_hbm, d_hbm, out_hbm, e0_v, e1_v, p_v, d0_v, d1_v,
                  tok_v, sem):
        wid = lax.axis_index("c") * 16 + lax.axis_index("s")

        def compute_toks(e_v, d_v):
            carry = jnp.float32(0.0)
            for c in range(_EPAD // 16):
                ch = e_v[pl.ds(c * 16, 16)]
                incl = plsc.cumsum(ch)
                p_v[pl.ds(c * 16, 16)] = (incl - ch) + carry
                carry = carry + jnp.sum(ch)
            dcarry = jnp.int32(0)
            for c in range(_TOKPAD // 16):
                dch = d_v[pl.ds(c * 16, 16)]
                ends = plsc.cumsum(dch) + dcarry
                starts = ends - dch
                dcarry = dcarry + jnp.sum(dch)
                pe = plsc.load_gather(p_v, [ends])
                ps = plsc.load_gather(p_v, [starts])
                dchf = dch.astype(jnp.float32)
                tok_v[pl.ds(c * 16, 16)] = jnp.where(
                    dch > 0, (pe - ps) / dchf, jnp.float32(0.0))

        @pl.when(wid < B)
        def _():
            cps = [pltpu.async_copy(e_hbm.at[0], e0_v, sem),
                   pltpu.async_copy(d_hbm.at[0], d0_v, sem),
                   pltpu.async_copy(e_hbm.at[wid], e1_v, sem),
                   pltpu.async_copy(d_hbm.at[wid], d1_v, sem)]
            for cp in cps:
                cp.wait()
            compute_toks(e0_v, d0_v)
            sv = jnp.zeros((16,), jnp.float32)
            nv = jnp.zeros((16,), jnp.float32)
            for c in range(_TOKPAD // 16):
                tc = tok_v[pl.ds(c * 16, 16)]
                sv = sv + tc
                nv = nv + (tc != 0.0).astype(jnp.float32)
            sb = jnp.full((16,), jnp.sum(sv), jnp.float32)
            nb = jnp.full((16,), jnp.sum(nv), jnp.float32)
            inv_avg = jnp.maximum(nb, 1.0) / sb

            compute_toks(e1_v, d1_v)
            for c in range(_TOKPAD // 16):
                tok_v[pl.ds(c * 16, 16)] = tok_v[pl.ds(c * 16, 16)] * inv_avg
            pltpu.sync_copy(tok_v, out_hbm.at[wid])

    return sc_kernel


def kernel(input_waves, durations, durations_lengths):
    B, T = input_waves.shape
    x3 = input_waves.reshape(B, T // _HOP, _HOP)
    xb = jnp.concatenate(
        [input_waves[:, 1:2 * _HOP + 1].reshape(B, 2, _HOP),
         input_waves[:, T - 2 * _HOP - 1:T - 1].reshape(B, 2, _HOP)],
        axis=1)
    n_tok = durations.shape[1]
    d64 = jnp.pad(durations.astype(jnp.int32),
                  ((0, 0), (0, _TOKPAD - n_tok)))
    e256 = pl.pallas_call(
        _energy_kernel,
        out_shape=jax.ShapeDtypeStruct((B, _EPAD), jnp.float32),
    )(x3, xb, jnp.asarray(_WMAT), jnp.asarray(_WMAT[:, ::-1].copy()))
    tok = _make_sc_segment_mean(B, n_tok)(e256, d64)
    return (tok[:, :n_tok, None], durations_lengths)


# speedup vs baseline: 1.0500x; 1.0500x over previous
import functools

import numpy as np
import jax
import jax.numpy as jnp
from jax import lax
from jax.experimental import pallas as pl
from jax.experimental.pallas import tpu as pltpu
from jax.experimental.pallas import tpu_sc as plsc

_N_FFT = 1024
_HOP = 256
_PAD = _N_FFT // 2
_EPAD = 256
_TOKPAD = 64

_n = np.arange(_N_FFT)
_w = (0.5 - 0.5 * np.cos(2.0 * np.pi * _n / _N_FFT)).astype(np.float32)
_walt = (_w * np.where(_n % 2 == 0, 1.0, -1.0)).astype(np.float32)
_wsq = (_w * _w).astype(np.float32)
_WMAT = np.zeros((16, _HOP), np.float32)
_WMAT[0:4] = _w.reshape(4, _HOP)
_WMAT[4:8] = _walt.reshape(4, _HOP)
_WMAT[8:12] = _wsq.reshape(4, _HOP)


def _energy_kernel(x_ref, xb_ref, w_ref, wr_ref, out_ref):
    x = x_ref[:]
    xb = xb_ref[:]
    w = w_ref[:]
    wr = wr_ref[:]
    B, n_interior, _ = x.shape
    n_chunks = n_interior + 4
    n_frames = n_chunks - 3

    xsq = x * x
    xbsq = xb * xb
    a = []
    for j in range(4):
        wj = w[j:j + 1, :].reshape(1, 1, _HOP)
        wjalt = w[j + 4:j + 5, :].reshape(1, 1, _HOP)
        wjsq = w[j + 8:j + 9, :].reshape(1, 1, _HOP)
        rj = wr[j:j + 1, :].reshape(1, 1, _HOP)
        rjalt = wr[j + 4:j + 5, :].reshape(1, 1, _HOP)
        rjsq = wr[j + 8:j + 9, :].reshape(1, 1, _HOP)
        ai = jnp.sum(x * wj, axis=-1)
        bi = jnp.sum(x * wjalt, axis=-1)
        ci = jnp.sum(xsq * wjsq, axis=-1)
        ab = jnp.sum(xb * rj, axis=-1)
        bb = jnp.sum(xb * rjalt, axis=-1)
        cb = jnp.sum(xbsq * rjsq, axis=-1)
        a.append((
            jnp.concatenate([ab[:, 1:2], ab[:, 0:1], ai,
                             ab[:, 3:4], ab[:, 2:3]], axis=1),
            jnp.concatenate([bb[:, 1:2], bb[:, 0:1], bi,
                             bb[:, 3:4], bb[:, 2:3]], axis=1),
            jnp.concatenate([cb[:, 1:2], cb[:, 0:1], ci,
                             cb[:, 3:4], cb[:, 2:3]], axis=1),
        ))

    s0 = a[0][0][:, 0:n_frames] + a[1][0][:, 1:n_frames + 1] \
        + a[2][0][:, 2:n_frames + 2] + a[3][0][:, 3:n_frames + 3]
    s1 = a[0][1][:, 0:n_frames] + a[1][1][:, 1:n_frames + 1] \
        + a[2][1][:, 2:n_frames + 2] + a[3][1][:, 3:n_frames + 3]
    s2 = a[0][2][:, 0:n_frames] + a[1][2][:, 1:n_frames + 1] \
        + a[2][2][:, 2:n_frames + 2] + a[3][2][:, 3:n_frames + 3]

    power = 0.5 * (_N_FFT * s2 + s0 * s0 + s1 * s1)
    e = jnp.sqrt(jnp.maximum(power, 1e-10))

    out_ref[:] = jnp.zeros((B, _EPAD), jnp.float32)
    out_ref[:, 0:n_frames] = e


def _make_sc_segment_mean(B, n_tok):
    mesh = plsc.VectorSubcoreMesh(core_axis_name="c", subcore_axis_name="s",
                                  num_cores=1)

    @functools.partial(
        pl.kernel, mesh=mesh,
        out_type=jax.ShapeDtypeStruct((B, _TOKPAD), jnp.float32),
        compiler_params=pltpu.CompilerParams(
            needs_layout_passes=False,
            disable_bounds_checks=True,
            disable_semaphore_checks=True,
        ),
        scratch_types=[
            pltpu.VMEM((_EPAD,), jnp.float32),
            pltpu.VMEM((_EPAD,), jnp.float32),
            pltpu.VMEM((_EPAD,), jnp.float32),
            pltpu.VMEM((_TOKPAD,), jnp.int32),
            pltpu.VMEM((_TOKPAD,), jnp.int32),
            pltpu.VMEM((_TOKPAD,), jnp.float32),
            pltpu.SemaphoreType.DMA,
        ],
    )
    def sc_kernel(e_hbm, d_hbm, out_hbm, e0_v, e1_v, p_v, d0_v, d1_v,
                  tok_v, sem):
        wid = lax.axis_index("c") * 16 + lax.axis_index("s")

        def compute_toks(e_v, d_v):
            carry = jnp.float32(0.0)
            for c in range(_EPAD // 16):
                ch = e_v[pl.ds(c * 16, 16)]
                incl = plsc.cumsum(ch)
                p_v[pl.ds(c * 16, 16)] = (incl - ch) + carry
                carry = carry + jnp.sum(ch)
            dcarry = jnp.int32(0)
            for c in range(_TOKPAD // 16):
                dch = d_v[pl.ds(c * 16, 16)]
                ends = plsc.cumsum(dch) + dcarry
                starts = ends - dch
                dcarry = dcarry + jnp.sum(dch)
                pe = plsc.load_gather(p_v, [ends])
                ps = plsc.load_gather(p_v, [starts])
                dchf = dch.astype(jnp.float32)
                tok_v[pl.ds(c * 16, 16)] = jnp.where(
                    dch > 0, (pe - ps) / dchf, jnp.float32(0.0))

        @pl.when(wid < B)
        def _():
            cps = [pltpu.async_copy(e_hbm.at[0], e0_v, sem),
                   pltpu.async_copy(d_hbm.at[0], d0_v, sem),
                   pltpu.async_copy(e_hbm.at[wid], e1_v, sem),
                   pltpu.async_copy(d_hbm.at[wid], d1_v, sem)]
            for cp in cps:
                cp.wait()
            compute_toks(e0_v, d0_v)
            sv = jnp.zeros((16,), jnp.float32)
            nv = jnp.zeros((16,), jnp.float32)
            for c in range(_TOKPAD // 16):
                tc = tok_v[pl.ds(c * 16, 16)]
                sv = sv + tc
                nv = nv + (tc != 0.0).astype(jnp.float32)
            sb = jnp.full((16,), jnp.sum(sv), jnp.float32)
            nb = jnp.full((16,), jnp.sum(nv), jnp.float32)
            inv_avg = jnp.maximum(nb, 1.0) / sb

            compute_toks(e1_v, d1_v)
            for c in range(_TOKPAD // 16):
                tok_v[pl.ds(c * 16, 16)] = tok_v[pl.ds(c * 16, 16)] * inv_avg
            pltpu.sync_copy(tok_v, out_hbm.at[wid])

    return sc_kernel


def kernel(input_waves, durations, durations_lengths):
    B, T = input_waves.shape
    x3 = input_waves.reshape(B, T // _HOP, _HOP)
    xb = jnp.concatenate(
        [input_waves[:, 1:2 * _HOP + 1].reshape(B, 2, _HOP),
         input_waves[:, T - 2 * _HOP - 1:T - 1].reshape(B, 2, _HOP)],
        axis=1)
    n_tok = durations.shape[1]
    d64 = jnp.pad(durations.astype(jnp.int32),
                  ((0, 0), (0, _TOKPAD - n_tok)))
    e256 = pl.pallas_call(
        _energy_kernel,
        out_shape=jax.ShapeDtypeStruct((B, _EPAD), jnp.float32),
    )(x3, xb, jnp.asarray(_WMAT), jnp.asarray(_WMAT[:, ::-1].copy()))
    tok = _make_sc_segment_mean(B, n_tok)(e256, d64)
    return (tok[:, :n_tok, None], durations_lengths)
